# baseline (speedup 1.0000x reference)
import jax
import jax.numpy as jnp
import numpy as np
from jax import lax
from jax.experimental import pallas as pl
from jax.experimental.pallas import tpu as pltpu
from jax.experimental.pallas import tpu_sc as plsc

BINS = 64
MARGIN = 0.4
N = 512 * 512
NSUB = 16
CH = N // NSUB
K_RANK = int(np.float32(0.95) * np.float32(N - 1))
Q_FRAC = float(np.float32(0.95) * np.float32(N - 1)) - K_RANK

_f32 = jnp.float32
_i32 = jnp.int32


def _body(pred_hbm, gt_hbm, out_hbm,
          gtbuf, pdbuf, cbuf, cnt, rbcnt, loc256, hist2, loc64, locv, rbmin,
          rbhist, s_cnt, s_min, s_hist):
    c = lax.axis_index("c")
    s = lax.axis_index("s")
    lane = lax.iota(_i32, 16)
    ones_i = jnp.full((16,), 1, _i32)
    zeros_i = jnp.full((16,), 0, _i32)
    zeros_f = jnp.full((16,), 0.0, _f32)

    @plsc.parallel_loop(0, 4096, step=16, unroll=4, carry=_i32(0))
    def _(kk, cy):
        cnt[pl.ds(kk, 16)] = zeros_i
        return cy

    @plsc.parallel_loop(0, 16 * BINS, step=16, unroll=4, carry=_i32(0))
    def _(kk, cy):
        hist2[pl.ds(kk, 16)] = zeros_f
        return cy

    loss_total = zeros_f
    for img in range(2):
        row = (2 * c + img) * NSUB + s
        pltpu.sync_copy(gt_hbm.at[row], gtbuf)
        pltpu.sync_copy(pred_hbm.at[row], pdbuf)

        inf_v = jnp.full((16,), jnp.inf, _f32)

        def combine_and_select(count_before):
            pltpu.sync_copy(loc256, s_cnt.at[pl.ds(s * 256, 256)])
            plsc.subcore_barrier()
            pltpu.sync_copy(s_cnt, rbcnt)
            r_loc = K_RANK - count_before

            def select(cc, carry):
                done, bin_, running, cb, cle = carry
                h = zeros_i
                for w in range(NSUB):
                    h = h + rbcnt[pl.ds(w * 256 + cc * 16, 16)]
                s_inc = plsc.cumsum(h)
                tot = jnp.sum(h)
                crossed = (running + s_inc) >= (r_loc + 1)
                anyc = jnp.sum(jnp.where(crossed, 1, 0)) > 0
                nfalse = jnp.sum(jnp.where(crossed, 0, 1))
                e_inc = jnp.sum(jnp.where(lane == nfalse, s_inc, 0))
                e_exc = e_inc - jnp.sum(jnp.where(lane == nfalse, h, 0))
                hit = jnp.logical_and(done == 0, anyc)
                bin_ = jnp.where(hit, cc * 16 + nfalse, bin_)
                cle = jnp.where(hit, running + e_inc, cle)
                cb = jnp.where(hit, running + e_exc, cb)
                done = jnp.where(hit, _i32(1), done)
                return done, bin_, running + tot, cb, cle

            _, bin_, _, cb, cle = lax.fori_loop(
                0, 16, select, (_i32(0), _i32(0), _i32(0), _i32(0), _i32(0)))
            plsc.subcore_barrier()
            return bin_, cb, cle

        def lane_reduce_zero(cc, cy):
            acc = zeros_i
            for l in range(16):
                acc = acc + cnt[pl.ds(l * 256 + cc, 16)]
                cnt[pl.ds(l * 256 + cc, 16)] = zeros_i
            loc256[pl.ds(cc, 16)] = acc
            return cy

        @plsc.parallel_loop(0, CH, step=16, unroll=8, carry=_i32(0))
        def _(kk, cy):
            v = gtbuf[pl.ds(kk, 16)]
            bits = lax.bitcast_convert_type(v, _i32)
            b = lax.shift_right_logical(bits, 24)
            plsc.addupdate_scatter(cnt, [lane * 256 + b], ones_i)
            return cy

        plsc.parallel_loop(0, 256, step=16, unroll=2, carry=_i32(0))(
            lane_reduce_zero)
        bin1, count_before, c_le = combine_and_select(_i32(0))

        @plsc.parallel_loop(0, CH, step=16, unroll=4,
                            carry=(_i32(0), inf_v))
        def cpr(kk, carry):
            off, mgt = carry
            v = gtbuf[pl.ds(kk, 16)]
            bits = lax.bitcast_convert_type(v, _i32)
            top = lax.shift_right_logical(bits, 24)
            m = top == bin1
            plsc.store_compressed(cbuf.at[pl.ds(off, 16)], v, mask=m)
            mgt = jnp.minimum(mgt, jnp.where(top > bin1, v, inf_v))
            return off + jnp.sum(jnp.where(m, 1, 0)), mgt

        off, mgt = cpr
        pad = jnp.full((16,), (bin1 ^ 1) << 24, _i32)
        cbuf[pl.ds(off, 16)] = lax.bitcast_convert_type(pad, _f32)
        nvec = lax.shift_right_logical(off + 15, 4)
        prefix = bin1

        for p in range(1, 4):
            sh = 24 - 8 * p

            def scan_c(kk, cy):
                v = cbuf[pl.ds(kk * 16, 16)]
                bits = lax.bitcast_convert_type(v, _i32)
                b = lax.shift_right_logical(bits, sh) & 255
                m = lax.shift_right_logical(bits, sh + 8) == prefix
                plsc.addupdate_scatter(cnt, [lane * 256 + b], ones_i, mask=m)
                return cy
            lax.fori_loop(0, nvec, scan_c, _i32(0))

            plsc.parallel_loop(0, 256, step=16, unroll=2, carry=_i32(0))(
                lane_reduce_zero)
            bin_, count_before, c_le = combine_and_select(count_before)
            prefix = (prefix << 8) | bin_

        vk = lax.bitcast_convert_type(jnp.full((16,), prefix, _i32), _f32)

        def mscan(kk, m):
            v = cbuf[pl.ds(kk * 16, 16)]
            bits = lax.bitcast_convert_type(v, _i32)
            ok = jnp.logical_and(lax.shift_right_logical(bits, 24) == bin1,
                                 v > vk)
            return jnp.minimum(m, jnp.where(ok, v, inf_v))
        mm = lax.fori_loop(0, nvec, mscan, mgt)
        locv[pl.ds(0, 16)] = mm
        pltpu.sync_copy(locv, s_min.at[pl.ds(s * 16, 16)])
        plsc.subcore_barrier()
        pltpu.sync_copy(s_min, rbmin)
        gmv = inf_v
        for w in range(NSUB):
            gmv = jnp.minimum(gmv, rbmin[pl.ds(w * 16, 16)])
        gm = jnp.min(gmv)
        have_next = jnp.full((16,), c_le, _i32) >= (K_RANK + 2)
        vnext = jnp.where(have_next, vk, jnp.full((16,), gm, _f32))
        max_val = vk + _f32(Q_FRAC) * (vnext - vk)
        scale = _f32(BINS) / max_val

        for ti, buf in ((0, pdbuf), (1, gtbuf)):
            @plsc.parallel_loop(0, CH, step=16, unroll=8, carry=_i32(0))
            def _(kk, cy):
                v = buf[pl.ds(kk, 16)]
                t = jnp.minimum(v * scale, _f32(65.0))
                j = t.astype(_i32)
                fr = t - j.astype(_f32)
                idx = lane * BINS + j
                plsc.addupdate_scatter(hist2, [idx], _f32(1.0) - fr,
                                       mask=j <= BINS - 1)
                plsc.addupdate_scatter(hist2, [idx + 1], fr,
                                       mask=j <= BINS - 2)
                return cy

            @plsc.parallel_loop(0, BINS, step=16, unroll=1, carry=_i32(0))
            def _(cc, cy):
                acc = zeros_f
                for l in range(16):
                    acc = acc + hist2[pl.ds(l * BINS + cc, 16)]
                    hist2[pl.ds(l * BINS + cc, 16)] = zeros_f
                loc64[pl.ds(cc, 16)] = acc
                return cy

            pltpu.sync_copy(loc64, s_hist.at[pl.ds((s * 2 + ti) * BINS, BINS)])
        plsc.subcore_barrier()

        @pl.when(s == 0)
        def _():
            pltpu.sync_copy(s_hist, rbhist)
            hp = []
            hg = []
            for cc in range(BINS // 16):
                accp = zeros_f
                accg = zeros_f
                for w in range(NSUB):
                    accp = accp + rbhist[pl.ds((w * 2 + 0) * BINS + cc * 16, 16)]
                    accg = accg + rbhist[pl.ds((w * 2 + 1) * BINS + cc * 16, 16)]
                hp.append(accp)
                hg.append(accg)
            psum = _f32(0.0)
            gsum = _f32(0.0)
            for cc in range(BINS // 16):
                psum = psum + jnp.sum(hp[cc])
                gsum = gsum + jnp.sum(hg[cc])
            ones_f = jnp.full((16,), 1.0, _f32)
            pinv = ones_f / jnp.full((16,), psum, _f32)
            ginv = ones_f / jnp.full((16,), gsum, _f32)
            li = _f32(0.0)
            for cc in range(BINS // 16):
                jbin = (lane + cc * 16).astype(_f32)
                wgt = jnp.exp(_f32(MARGIN) * jbin * _f32(1.0 / BINS))
                diff = jnp.abs(hp[cc] * pinv * wgt - hg[cc] * ginv * wgt)
                li = li + jnp.sum(diff)
            locv[pl.ds(0, 16)] = jnp.full((16,), li * _f32(1.0 / BINS), _f32)
        plsc.subcore_barrier()
        loss_total = loss_total + locv[pl.ds(0, 16)]

    @pl.when(s == 0)
    def _():
        locv[pl.ds(0, 16)] = loss_total
        pltpu.sync_copy(locv, out_hbm.at[c])


def kernel(pred_grad, gt_grad):
    pred2 = pred_grad.reshape(4 * NSUB, CH)
    gt2 = gt_grad.reshape(4 * NSUB, CH)
    mesh = plsc.VectorSubcoreMesh(core_axis_name="c", subcore_axis_name="s")
    k = pl.kernel(
        _body,
        out_type=jax.ShapeDtypeStruct((2, 16), _f32),
        mesh=mesh,
        compiler_params=pltpu.CompilerParams(needs_layout_passes=False),
        scratch_types=[
            pltpu.VMEM((CH,), _f32),
            pltpu.VMEM((CH,), _f32),
            pltpu.VMEM((CH + 16,), _f32),
            pltpu.VMEM((4096,), _i32),
            pltpu.VMEM((4096,), _i32),
            pltpu.VMEM((256,), _i32),
            pltpu.VMEM((16 * BINS,), _f32),
            pltpu.VMEM((BINS,), _f32),
            pltpu.VMEM((16,), _f32),
            pltpu.VMEM((256,), _f32),
            pltpu.VMEM((2 * NSUB * BINS,), _f32),
            pltpu.VMEM_SHARED((NSUB * 256,), _i32),
            pltpu.VMEM_SHARED((NSUB * 16,), _f32),
            pltpu.VMEM_SHARED((2 * NSUB * BINS,), _f32),
        ],
    )
    out = k(pred2, gt2)
    return (out[0, 0] + out[1, 0]) * _f32(0.25)


# speedup vs baseline: 1.2955x; 1.2955x over previous
import jax
import jax.numpy as jnp
import numpy as np
from jax import lax
from jax.experimental import pallas as pl
from jax.experimental.pallas import tpu as pltpu
from jax.experimental.pallas import tpu_sc as plsc

BINS = 64
MARGIN = 0.4
N = 512 * 512
NSUB = 16
CH = N // NSUB
K_RANK = int(np.float32(0.95) * np.float32(N - 1))
Q_FRAC = float(np.float32(0.95) * np.float32(N - 1)) - K_RANK

_f32 = jnp.float32
_i32 = jnp.int32

_W64 = np.exp(np.float32(MARGIN) * np.arange(BINS, dtype=np.float32)
              / np.float32(BINS)).astype(np.float32)


def _body(pred_hbm, gt_hbm, wts_hbm, out_hbm,
          gtbuf, pdbuf, cnt, rbcnt, loc256, hist2, loc64, locv, wbuf, rbmin,
          rbhist, s_cnt, s_min, s_hist):
    c = lax.axis_index("c")
    s = lax.axis_index("s")
    lane = lax.iota(_i32, 16)
    ones_i = jnp.full((16,), 1, _i32)
    zeros_i = jnp.full((16,), 0, _i32)
    zeros_f = jnp.full((16,), 0.0, _f32)

    @plsc.parallel_loop(0, 4096, step=16, unroll=4, carry=_i32(0))
    def _(kk, cy):
        cnt[pl.ds(kk, 16)] = zeros_i
        return cy

    @plsc.parallel_loop(0, 16 * BINS, step=16, unroll=4, carry=_i32(0))
    def _(kk, cy):
        hist2[pl.ds(kk, 16)] = zeros_f
        return cy

    @pl.when(s == 0)
    def _():
        pltpu.sync_copy(wts_hbm, wbuf)

    loss_total = zeros_f
    for img in range(2):
        row = (2 * c + img) * NSUB + s
        pltpu.sync_copy(gt_hbm.at[row], gtbuf)
        pltpu.sync_copy(pred_hbm.at[row], pdbuf)

        inf_v = jnp.full((16,), jnp.inf, _f32)

        def combine_and_select(count_before):
            pltpu.sync_copy(loc256, s_cnt.at[pl.ds(s * 256, 256)])
            plsc.subcore_barrier()
            pltpu.sync_copy(s_cnt, rbcnt)
            r_loc = K_RANK - count_before

            def select(cc, carry):
                done, bin_, running, cb, cle = carry
                h = zeros_i
                for w in range(NSUB):
                    h = h + rbcnt[pl.ds(w * 256 + cc * 16, 16)]
                s_inc = plsc.cumsum(h)
                tot = jnp.sum(h)
                crossed = (running + s_inc) >= (r_loc + 1)
                anyc = jnp.sum(jnp.where(crossed, 1, 0)) > 0
                nfalse = jnp.sum(jnp.where(crossed, 0, 1))
                e_inc = jnp.sum(jnp.where(lane == nfalse, s_inc, 0))
                e_exc = e_inc - jnp.sum(jnp.where(lane == nfalse, h, 0))
                hit = jnp.logical_and(done == 0, anyc)
                bin_ = jnp.where(hit, cc * 16 + nfalse, bin_)
                cle = jnp.where(hit, running + e_inc, cle)
                cb = jnp.where(hit, running + e_exc, cb)
                done = jnp.where(hit, _i32(1), done)
                return done, bin_, running + tot, cb, cle

            _, bin_, _, cb, cle = lax.fori_loop(
                0, 16, select, (_i32(0), _i32(0), _i32(0), _i32(0), _i32(0)))
            plsc.subcore_barrier()
            return bin_, cb, cle

        def lane_reduce_zero(cc, cy):
            acc = zeros_i
            for l in range(16):
                acc = acc + cnt[pl.ds(l * 256 + cc, 16)]
                cnt[pl.ds(l * 256 + cc, 16)] = zeros_i
            loc256[pl.ds(cc, 16)] = acc
            return cy

        prefix = _i32(0)
        count_before = _i32(0)
        c_le = _i32(0)
        for p in range(4):
            sh = 24 - 8 * p

            @plsc.parallel_loop(0, CH, step=16, unroll=8, carry=_i32(0))
            def _(kk, cy):
                v = gtbuf[pl.ds(kk, 16)]
                bits = lax.bitcast_convert_type(v, _i32)
                b = lax.shift_right_logical(bits, sh) & 255
                idx = lane * 256 + b
                if p == 0:
                    plsc.addupdate_scatter(cnt, [idx], ones_i)
                else:
                    m = lax.shift_right_logical(bits, sh + 8) == prefix
                    plsc.addupdate_scatter(cnt, [idx], ones_i, mask=m)
                return cy

            plsc.parallel_loop(0, 256, step=16, unroll=2, carry=_i32(0))(
                lane_reduce_zero)
            bin_, count_before, c_le = combine_and_select(count_before)
            prefix = (prefix << 8) | bin_ if p > 0 else bin_

        vk = lax.bitcast_convert_type(jnp.full((16,), prefix, _i32), _f32)

        @plsc.parallel_loop(0, CH, step=16, unroll=8, carry=inf_v)
        def mm(kk, m):
            v = gtbuf[pl.ds(kk, 16)]
            return jnp.minimum(m, jnp.where(v > vk, v, inf_v))
        locv[pl.ds(0, 16)] = mm
        pltpu.sync_copy(locv, s_min.at[pl.ds(s * 16, 16)])
        plsc.subcore_barrier()
        pltpu.sync_copy(s_min, rbmin)
        gmv = inf_v
        for w in range(NSUB):
            gmv = jnp.minimum(gmv, rbmin[pl.ds(w * 16, 16)])
        gm = jnp.min(gmv)
        have_next = jnp.full((16,), c_le, _i32) >= (K_RANK + 2)
        vnext = jnp.where(have_next, vk, jnp.full((16,), gm, _f32))
        max_val = vk + _f32(Q_FRAC) * (vnext - vk)
        r0 = jnp.full((16,), 1.0, _f32) / max_val
        r0 = r0 * (_f32(2.0) - max_val * r0)
        r0 = r0 * (_f32(2.0) - max_val * r0)
        scale = _f32(BINS) * r0

        for ti, buf in ((0, pdbuf), (1, gtbuf)):
            @plsc.parallel_loop(0, CH, step=16, unroll=8, carry=_i32(0))
            def _(kk, cy):
                v = buf[pl.ds(kk, 16)]
                t = jnp.minimum(v * scale, _f32(65.0))
                j = t.astype(_i32)
                fr = t - j.astype(_f32)
                idx = lane * BINS + j
                plsc.addupdate_scatter(hist2, [idx], _f32(1.0) - fr,
                                       mask=j <= BINS - 1)
                plsc.addupdate_scatter(hist2, [idx + 1], fr,
                                       mask=j <= BINS - 2)
                return cy

            @plsc.parallel_loop(0, BINS, step=16, unroll=1, carry=_i32(0))
            def _(cc, cy):
                acc = zeros_f
                for l in range(16):
                    acc = acc + hist2[pl.ds(l * BINS + cc, 16)]
                    hist2[pl.ds(l * BINS + cc, 16)] = zeros_f
                loc64[pl.ds(cc, 16)] = acc
                return cy

            pltpu.sync_copy(loc64, s_hist.at[pl.ds((s * 2 + ti) * BINS, BINS)])
        plsc.subcore_barrier()

        @pl.when(s == 0)
        def _():
            pltpu.sync_copy(s_hist, rbhist)
            hp = []
            hg = []
            for cc in range(BINS // 16):
                accp = zeros_f
                accg = zeros_f
                for w in range(NSUB):
                    accp = accp + rbhist[pl.ds((w * 2 + 0) * BINS + cc * 16, 16)]
                    accg = accg + rbhist[pl.ds((w * 2 + 1) * BINS + cc * 16, 16)]
                hp.append(accp)
                hg.append(accg)
            psum = _f32(0.0)
            gsum = _f32(0.0)
            for cc in range(BINS // 16):
                psum = psum + jnp.sum(hp[cc])
                gsum = gsum + jnp.sum(hg[cc])
            pv = jnp.full((16,), psum, _f32)
            gv = jnp.full((16,), gsum, _f32)
            li = _f32(0.0)
            for cc in range(BINS // 16):
                wgt = wbuf[pl.ds(cc * 16, 16)]
                diff = jnp.abs(hp[cc] * gv - hg[cc] * pv) * wgt
                li = li + jnp.sum(diff)
            pg = pv * gv
            q0 = jnp.full((16,), 1.0, _f32) / pg
            q0 = q0 * (_f32(2.0) - pg * q0)
            q0 = q0 * (_f32(2.0) - pg * q0)
            lv = jnp.full((16,), li * _f32(1.0 / BINS), _f32) * q0
            locv[pl.ds(0, 16)] = lv
        plsc.subcore_barrier()
        loss_total = loss_total + locv[pl.ds(0, 16)]

    @pl.when(s == 0)
    def _():
        locv[pl.ds(0, 16)] = loss_total
        pltpu.sync_copy(locv, out_hbm.at[c])


def kernel(pred_grad, gt_grad):
    pred2 = pred_grad.reshape(4 * NSUB, CH)
    gt2 = gt_grad.reshape(4 * NSUB, CH)
    mesh = plsc.VectorSubcoreMesh(core_axis_name="c", subcore_axis_name="s")
    k = pl.kernel(
        _body,
        out_type=jax.ShapeDtypeStruct((2, 16), _f32),
        mesh=mesh,
        compiler_params=pltpu.CompilerParams(needs_layout_passes=False),
        scratch_types=[
            pltpu.VMEM((CH,), _f32),
            pltpu.VMEM((CH,), _f32),
            pltpu.VMEM((4096,), _i32),
            pltpu.VMEM((4096,), _i32),
            pltpu.VMEM((256,), _i32),
            pltpu.VMEM((16 * BINS,), _f32),
            pltpu.VMEM((BINS,), _f32),
            pltpu.VMEM((16,), _f32),
            pltpu.VMEM((BINS,), _f32),
            pltpu.VMEM((256,), _f32),
            pltpu.VMEM((2 * NSUB * BINS,), _f32),
            pltpu.VMEM_SHARED((NSUB * 256,), _i32),
            pltpu.VMEM_SHARED((NSUB * 16,), _f32),
            pltpu.VMEM_SHARED((2 * NSUB * BINS,), _f32),
        ],
    )
    out = k(pred2, gt2, jnp.asarray(_W64))
    return (out[0, 0] + out[1, 0]) * _f32(0.25)


# speedup vs baseline: 1.4548x; 1.1229x over previous
import jax
import jax.numpy as jnp
import numpy as np
from jax import lax
from jax.experimental import pallas as pl
from jax.experimental.pallas import tpu as pltpu
from jax.experimental.pallas import tpu_sc as plsc

BINS = 64
MARGIN = 0.4
N = 512 * 512
NSUB = 16
CH = N // NSUB
K_RANK = int(np.float32(0.95) * np.float32(N - 1))
Q_FRAC = float(np.float32(0.95) * np.float32(N - 1)) - K_RANK

_f32 = jnp.float32
_i32 = jnp.int32

_W64 = np.exp(np.float32(MARGIN) * np.arange(BINS, dtype=np.float32)
              / np.float32(BINS)).astype(np.float32)


def _body(pred_hbm, gt_hbm, wts_hbm, out_hbm,
          gt0, gt1, pd0, pd1, cnt, hist2, loc2d, rb2d, h2d, rbh2d,
          zacc, locv, minv, wbuf, rbmin,
          s_acc, s_hacc, s_min,
          sem_g0, sem_g1, sem_p0, sem_p1):
    c = lax.axis_index("c")
    s = lax.axis_index("s")
    lane = lax.iota(_i32, 16)
    ones_i = jnp.full((16,), 1, _i32)
    zeros_i = jnp.full((16,), 0, _i32)
    zeros_f = jnp.full((16,), 0.0, _f32)
    inf_v = jnp.full((16,), jnp.inf, _f32)

    row0 = (2 * c + 0) * NSUB + s
    row1 = (2 * c + 1) * NSUB + s
    cp_g0 = pltpu.async_copy(gt_hbm.at[row0], gt0, sem_g0)
    cp_g1 = pltpu.async_copy(gt_hbm.at[row1], gt1, sem_g1)
    cp_p0 = pltpu.async_copy(pred_hbm.at[row0], pd0, sem_p0)
    cp_p1 = pltpu.async_copy(pred_hbm.at[row1], pd1, sem_p1)

    @plsc.parallel_loop(0, 4096, step=16, unroll=4, carry=_i32(0))
    def _(kk, cy):
        cnt[pl.ds(kk, 16)] = zeros_i
        hist2[pl.ds(kk, 16)] = zeros_f
        return cy

    @plsc.parallel_loop(0, 128, step=1, unroll=4, carry=_i32(0))
    def _(rr, cy):
        zacc[rr] = zeros_i
        return cy

    for rr in range(16):
        h2d[rr] = zeros_f

    @pl.when(s == 0)
    def _():
        pltpu.sync_copy(zacc, s_acc)
        pltpu.sync_copy(h2d, s_hacc)
        pltpu.sync_copy(wts_hbm, wbuf)
    plsc.subcore_barrier()

    loss_total = zeros_f
    for img in range(2):
        gbuf = (gt0, gt1)[img]
        pbuf = (pd0, pd1)[img]
        (cp_g0, cp_g1)[img].wait()

        prefix = _i32(0)
        count_before = _i32(0)
        c_le = _i32(0)
        nminb = _i32(256)
        for p in range(4):
            sh = 24 - 8 * p

            @plsc.parallel_loop(0, CH, step=16, unroll=8, carry=_i32(0))
            def _(kk, cy):
                v = gbuf[pl.ds(kk, 16)]
                bits = lax.bitcast_convert_type(v, _i32)
                if p == 0:
                    b = lax.shift_right_logical(bits, 24)
                    plsc.addupdate_scatter(cnt, [lane * 256 + b], ones_i)
                else:
                    b = lax.shift_right_logical(bits, sh) & 255
                    m = lax.shift_right_logical(bits, sh + 8) == prefix
                    plsc.addupdate_scatter(cnt, [lane * 256 + b], ones_i,
                                           mask=m)
                return cy

            @plsc.parallel_loop(0, 16, step=1, unroll=2, carry=_i32(0))
            def _(cc, cy):
                acc = zeros_i
                for l in range(16):
                    acc = acc + cnt[pl.ds(l * 256 + cc * 16, 16)]
                    cnt[pl.ds(l * 256 + cc * 16, 16)] = zeros_i
                loc2d[cc] = acc
                return cy

            slot = (img * 4 + p) * 16
            pltpu.sync_copy(loc2d, s_acc.at[slot + lane], add=True)
            plsc.subcore_barrier()
            pltpu.sync_copy(s_acc.at[pl.ds(slot, 16)], rb2d)

            r_loc = K_RANK - count_before

            def select(cc, carry):
                done, bin_, running, cb, cle = carry
                h = rb2d[cc]
                s_inc = plsc.cumsum(h)
                tot = jnp.sum(h)
                crossed = (running + s_inc) >= (r_loc + 1)
                anyc = jnp.sum(jnp.where(crossed, 1, 0)) > 0
                nfalse = jnp.sum(jnp.where(crossed, 0, 1))
                e_inc = jnp.sum(jnp.where(lane == nfalse, s_inc, 0))
                e_exc = e_inc - jnp.sum(jnp.where(lane == nfalse, h, 0))
                hit = jnp.logical_and(done == 0, anyc)
                bin_ = jnp.where(hit, cc * 16 + nfalse, bin_)
                cle = jnp.where(hit, running + e_inc, cle)
                cb = jnp.where(hit, running + e_exc, cb)
                done = jnp.where(hit, _i32(1), done)
                return done, bin_, running + tot, cb, cle

            _, bin_, _, count_before, c_le = lax.fori_loop(
                0, 16, select, (_i32(0), _i32(0), _i32(0), _i32(0), _i32(0)))

            if p == 3:
                def nxt(cc, nm):
                    h = rb2d[cc]
                    idxb = cc * 16 + lane
                    cand = jnp.where(
                        jnp.logical_and(idxb > bin_, h > 0), idxb, 256)
                    return jnp.minimum(nm, jnp.min(cand))

                nminb = lax.fori_loop(0, 16, nxt, _i32(256))
            prefix = (prefix << 8) | bin_ if p > 0 else bin_

        vk = lax.bitcast_convert_type(jnp.full((16,), prefix, _i32), _f32)

        scan_needed = jnp.logical_and(c_le < K_RANK + 2, nminb >= 256)

        @pl.when(scan_needed)
        def _():
            @plsc.parallel_loop(0, CH, step=16, unroll=8, carry=inf_v)
            def mm(kk, m):
                v = gbuf[pl.ds(kk, 16)]
                return jnp.minimum(m, jnp.where(v > vk, v, inf_v))
            minv[pl.ds(0, 16)] = mm
            pltpu.sync_copy(minv, s_min.at[pl.ds(s * 16, 16)])
        plsc.subcore_barrier()

        @pl.when(scan_needed)
        def _():
            pltpu.sync_copy(s_min, rbmin)
            gmv = inf_v
            for w in range(NSUB):
                gmv = jnp.minimum(gmv, rbmin[pl.ds(w * 16, 16)])
            minv[pl.ds(0, 16)] = jnp.full((16,), jnp.min(gmv), _f32)

        vnext_pfx = lax.bitcast_convert_type(
            jnp.full((16,), prefix + (nminb - (prefix & 255)), _i32), _f32)
        have_dup = jnp.full((16,), c_le, _i32) >= (K_RANK + 2)
        in_pfx = jnp.full((16,), nminb, _i32) <= 255
        vnext = jnp.where(have_dup, vk,
                          jnp.where(in_pfx, vnext_pfx, minv[pl.ds(0, 16)]))
        max_val = vk + _f32(Q_FRAC) * (vnext - vk)
        r0 = jnp.full((16,), 1.0, _f32) / max_val
        r0 = r0 * (_f32(2.0) - max_val * r0)
        r0 = r0 * (_f32(2.0) - max_val * r0)
        scale = _f32(BINS) * r0

        (cp_p0, cp_p1)[img].wait()

        @plsc.parallel_loop(0, CH, step=16, unroll=4, carry=_i32(0))
        def _(kk, cy):
            for ti, buf in ((0, pbuf), (1, gbuf)):
                v = buf[pl.ds(kk, 16)]
                t = jnp.minimum(v * scale, _f32(65.0))
                j = t.astype(_i32)
                fr = t - j.astype(_f32)
                idx = lane * 256 + (img * 128 + ti * 64) + j
                plsc.addupdate_scatter(hist2, [idx], _f32(1.0) - fr,
                                       mask=j <= BINS - 1)
                plsc.addupdate_scatter(hist2, [idx + 1], fr,
                                       mask=j <= BINS - 2)
            return cy

    @plsc.parallel_loop(0, 16, step=1, unroll=2, carry=_i32(0))
    def _(cc, cy):
        acc = zeros_f
        for l in range(16):
            acc = acc + hist2[pl.ds(l * 256 + cc * 16, 16)]
        h2d[cc] = acc
        return cy

    pltpu.sync_copy(h2d, s_hacc.at[lane], add=True)
    plsc.subcore_barrier()

    @pl.when(s == 0)
    def _():
        pltpu.sync_copy(s_hacc, rbh2d)
        loss = zeros_f
        for img in range(2):
            hp = [rbh2d[img * 8 + cc] for cc in range(BINS // 16)]
            hg = [rbh2d[img * 8 + 4 + cc] for cc in range(BINS // 16)]
            psum = _f32(0.0)
            gsum = _f32(0.0)
            for cc in range(BINS // 16):
                psum = psum + jnp.sum(hp[cc])
                gsum = gsum + jnp.sum(hg[cc])
            pv = jnp.full((16,), psum, _f32)
            gv = jnp.full((16,), gsum, _f32)
            li = _f32(0.0)
            for cc in range(BINS // 16):
                wgt = wbuf[pl.ds(cc * 16, 16)]
                diff = jnp.abs(hp[cc] * gv - hg[cc] * pv) * wgt
                li = li + jnp.sum(diff)
            pg = pv * gv
            q0 = jnp.full((16,), 1.0, _f32) / pg
            q0 = q0 * (_f32(2.0) - pg * q0)
            q0 = q0 * (_f32(2.0) - pg * q0)
            loss = loss + jnp.full((16,), li * _f32(1.0 / BINS), _f32) * q0
        locv[pl.ds(0, 16)] = loss
        pltpu.sync_copy(locv, out_hbm.at[c])


def kernel(pred_grad, gt_grad):
    pred2 = pred_grad.reshape(4 * NSUB, CH)
    gt2 = gt_grad.reshape(4 * NSUB, CH)
    mesh = plsc.VectorSubcoreMesh(core_axis_name="c", subcore_axis_name="s")
    k = pl.kernel(
        _body,
        out_type=jax.ShapeDtypeStruct((2, 16), _f32),
        mesh=mesh,
        compiler_params=pltpu.CompilerParams(needs_layout_passes=False),
        scratch_types=[
            pltpu.VMEM((CH,), _f32),
            pltpu.VMEM((CH,), _f32),
            pltpu.VMEM((CH,), _f32),
            pltpu.VMEM((CH,), _f32),
            pltpu.VMEM((4096,), _i32),
            pltpu.VMEM((4096,), _f32),
            pltpu.VMEM((16, 16), _i32),
            pltpu.VMEM((16, 16), _i32),
            pltpu.VMEM((16, 16), _f32),
            pltpu.VMEM((16, 16), _f32),
            pltpu.VMEM((128, 16), _i32),
            pltpu.VMEM((16,), _f32),
            pltpu.VMEM((16,), _f32),
            pltpu.VMEM((BINS,), _f32),
            pltpu.VMEM((256,), _f32),
            pltpu.VMEM_SHARED((128, 16), _i32),
            pltpu.VMEM_SHARED((16, 16), _f32),
            pltpu.VMEM_SHARED((NSUB * 16,), _f32),
            pltpu.SemaphoreType.DMA,
            pltpu.SemaphoreType.DMA,
            pltpu.SemaphoreType.DMA,
            pltpu.SemaphoreType.DMA,
        ],
    )
    out = k(pred2, gt2, jnp.asarray(_W64))
    return (out[0, 0] + out[1, 0]) * _f32(0.25)


# speedup vs baseline: 1.4912x; 1.0250x over previous
import jax
import jax.numpy as jnp
import numpy as np
from jax import lax
from jax.experimental import pallas as pl
from jax.experimental.pallas import tpu as pltpu
from jax.experimental.pallas import tpu_sc as plsc

BINS = 64
MARGIN = 0.4
N = 512 * 512
NSUB = 16
CH = N // NSUB
K_RANK = int(np.float32(0.95) * np.float32(N - 1))
Q_FRAC = float(np.float32(0.95) * np.float32(N - 1)) - K_RANK

_f32 = jnp.float32
_i32 = jnp.int32

_W64 = np.exp(np.float32(MARGIN) * np.arange(BINS, dtype=np.float32)
              / np.float32(BINS)).astype(np.float32)


def _body(pred_hbm, gt_hbm, wts_hbm, out_hbm,
          gt0, gt1, pd0, pd1, cnt, hist2, loc2d, rb2d, h2d, rbh2d,
          zacc, locv, minv, wbuf, rbmin,
          s_acc, s_hacc, s_min,
          sem_g0, sem_g1, sem_p0, sem_p1):
    c = lax.axis_index("c")
    s = lax.axis_index("s")
    lane = lax.iota(_i32, 16)
    ones_i = jnp.full((16,), 1, _i32)
    zeros_i = jnp.full((16,), 0, _i32)
    zeros_f = jnp.full((16,), 0.0, _f32)
    inf_v = jnp.full((16,), jnp.inf, _f32)
    lane512 = lane * 512

    row0 = (2 * c + 0) * NSUB + s
    row1 = (2 * c + 1) * NSUB + s
    cp_g0 = pltpu.async_copy(gt_hbm.at[row0], gt0, sem_g0)
    cp_g1 = pltpu.async_copy(gt_hbm.at[row1], gt1, sem_g1)
    cp_p0 = pltpu.async_copy(pred_hbm.at[row0], pd0, sem_p0)
    cp_p1 = pltpu.async_copy(pred_hbm.at[row1], pd1, sem_p1)

    @plsc.parallel_loop(0, 4096, step=16, unroll=4, carry=_i32(0))
    def _(kk, cy):
        cnt[pl.ds(kk, 16)] = zeros_i
        cnt[pl.ds(4096 + kk, 16)] = zeros_i
        hist2[pl.ds(kk, 16)] = zeros_f
        return cy

    @plsc.parallel_loop(0, 128, step=1, unroll=4, carry=_i32(0))
    def _(rr, cy):
        zacc[rr] = zeros_i
        return cy

    for rr in range(16):
        h2d[rr] = zeros_f

    @pl.when(s == 0)
    def _():
        pltpu.sync_copy(zacc, s_acc)
        pltpu.sync_copy(h2d, s_hacc)
        pltpu.sync_copy(wts_hbm, wbuf)
    plsc.subcore_barrier()

    cp_g0.wait()
    cp_g1.wait()

    prefix = [_i32(0), _i32(0)]
    count_before = [_i32(0), _i32(0)]
    c_le = [_i32(0), _i32(0)]
    nminb = [_i32(256), _i32(256)]
    binlow = [_i32(0), _i32(0)]
    for p in range(4):
        sh = 24 - 8 * p
        pfx0, pfx1 = prefix[0], prefix[1]

        @plsc.parallel_loop(0, CH, step=16, unroll=4, carry=_i32(0))
        def _(kk, cy):
            for gbuf, off, pfx in ((gt0, 0, pfx0), (gt1, 256, pfx1)):
                v = gbuf[pl.ds(kk, 16)]
                bits = lax.bitcast_convert_type(v, _i32)
                if p == 0:
                    b = lax.shift_right_logical(bits, 24)
                    plsc.addupdate_scatter(cnt, [lane512 + (off + b)], ones_i)
                else:
                    b = lax.shift_right_logical(bits, sh) & 255
                    m = lax.shift_right_logical(bits, sh + 8) == pfx
                    plsc.addupdate_scatter(cnt, [lane512 + (off + b)], ones_i,
                                           mask=m)
            return cy

        @plsc.parallel_loop(0, 32, step=1, unroll=2, carry=_i32(0))
        def _(cc, cy):
            acc = zeros_i
            for l in range(16):
                acc = acc + cnt[pl.ds(l * 512 + cc * 16, 16)]
                cnt[pl.ds(l * 512 + cc * 16, 16)] = zeros_i
            loc2d[cc] = acc
            return cy

        slot = p * 32
        pltpu.sync_copy(loc2d.at[pl.ds(0, 16)], s_acc.at[slot + lane],
                        add=True)
        pltpu.sync_copy(loc2d.at[pl.ds(16, 16)], s_acc.at[slot + 16 + lane],
                        add=True)
        plsc.subcore_barrier()
        pltpu.sync_copy(s_acc.at[pl.ds(slot, 32)], rb2d)

        for img in range(2):
            r_loc = K_RANK - count_before[img]
            base = img * 16

            def select(cc, carry, base=base, r_loc=r_loc):
                done, bin_, running, cb, cle = carry
                h = rb2d[base + cc]
                s_inc = plsc.cumsum(h)
                tot = jnp.sum(h)
                crossed = (running + s_inc) >= (r_loc + 1)
                anyc = jnp.sum(jnp.where(crossed, 1, 0)) > 0
                nfalse = jnp.sum(jnp.where(crossed, 0, 1))
                e_inc = jnp.sum(jnp.where(lane == nfalse, s_inc, 0))
                e_exc = e_inc - jnp.sum(jnp.where(lane == nfalse, h, 0))
                hit = jnp.logical_and(done == 0, anyc)
                bin_ = jnp.where(hit, cc * 16 + nfalse, bin_)
                cle = jnp.where(hit, running + e_inc, cle)
                cb = jnp.where(hit, running + e_exc, cb)
                done = jnp.where(hit, _i32(1), done)
                return done, bin_, running + tot, cb, cle

            _, bin_, _, cb, cle = lax.fori_loop(
                0, 16, select, (_i32(0), _i32(0), _i32(0), _i32(0), _i32(0)))
            count_before[img] = cb
            c_le[img] = cle
            binlow[img] = bin_

            if p == 3:
                def nxt(cc, nm, base=base, bin_=bin_):
                    h = rb2d[base + cc]
                    idxb = cc * 16 + lane
                    cand = jnp.where(
                        jnp.logical_and(idxb > bin_, h > 0), idxb, 256)
                    return jnp.minimum(nm, jnp.min(cand))

                nminb[img] = lax.fori_loop(0, 16, nxt, _i32(256))
            prefix[img] = (prefix[img] << 8) | bin_ if p > 0 else bin_

    vk = [lax.bitcast_convert_type(jnp.full((16,), prefix[i], _i32), _f32)
          for i in range(2)]
    scan_needed = [jnp.logical_and(c_le[i] < K_RANK + 2, nminb[i] >= 256)
                   for i in range(2)]

    for img in range(2):
        gbuf = (gt0, gt1)[img]
        vki = vk[img]

        @pl.when(scan_needed[img])
        def _(gbuf=gbuf, vki=vki, img=img):
            @plsc.parallel_loop(0, CH, step=16, unroll=8, carry=inf_v)
            def mm(kk, m):
                v = gbuf[pl.ds(kk, 16)]
                return jnp.minimum(m, jnp.where(v > vki, v, inf_v))
            minv[img] = mm
            pltpu.sync_copy(minv.at[img],
                            s_min.at[pl.ds(img * 256 + s * 16, 16)])
    plsc.subcore_barrier()

    for img in range(2):
        @pl.when(scan_needed[img])
        def _(img=img):
            pltpu.sync_copy(s_min.at[pl.ds(img * 256, 256)], rbmin)
            gmv = inf_v
            for w in range(NSUB):
                gmv = jnp.minimum(gmv, rbmin[pl.ds(w * 16, 16)])
            minv[img] = jnp.full((16,), jnp.min(gmv), _f32)

    scales = []
    for img in range(2):
        vnext_pfx = lax.bitcast_convert_type(
            jnp.full((16,), prefix[img] + (nminb[img] - binlow[img]), _i32),
            _f32)
        have_dup = jnp.full((16,), c_le[img], _i32) >= (K_RANK + 2)
        in_pfx = jnp.full((16,), nminb[img], _i32) <= 255
        vnext = jnp.where(have_dup, vk[img],
                          jnp.where(in_pfx, vnext_pfx, minv[img]))
        max_val = vk[img] + _f32(Q_FRAC) * (vnext - vk[img])
        r0 = jnp.full((16,), 1.0, _f32) / max_val
        r0 = r0 * (_f32(2.0) - max_val * r0)
        r0 = r0 * (_f32(2.0) - max_val * r0)
        scales.append(_f32(BINS) * r0)

    cp_p0.wait()
    cp_p1.wait()
    lane256 = lane * 256
    sc0, sc1 = scales

    @plsc.parallel_loop(0, CH, step=16, unroll=2, carry=_i32(0))
    def _(kk, cy):
        for buf, off, sc in ((pd0, 0, sc0), (gt0, 64, sc0),
                             (pd1, 128, sc1), (gt1, 192, sc1)):
            v = buf[pl.ds(kk, 16)]
            t = v * sc
            j = t.astype(_i32)
            fr = t - j.astype(_f32)
            idx = lane256 + (off + j)
            plsc.addupdate_scatter(hist2, [idx], _f32(1.0) - fr,
                                   mask=t < _f32(64.0))
            plsc.addupdate_scatter(hist2, [idx + 1], fr,
                                   mask=t < _f32(63.0))
        return cy

    @plsc.parallel_loop(0, 16, step=1, unroll=2, carry=_i32(0))
    def _(cc, cy):
        acc = zeros_f
        for l in range(16):
            acc = acc + hist2[pl.ds(l * 256 + cc * 16, 16)]
        h2d[cc] = acc
        return cy

    pltpu.sync_copy(h2d, s_hacc.at[lane], add=True)
    plsc.subcore_barrier()

    @pl.when(s == 0)
    def _():
        pltpu.sync_copy(s_hacc, rbh2d)
        loss = zeros_f
        for img in range(2):
            hp = [rbh2d[img * 8 + cc] for cc in range(BINS // 16)]
            hg = [rbh2d[img * 8 + 4 + cc] for cc in range(BINS // 16)]
            psum = _f32(0.0)
            gsum = _f32(0.0)
            for cc in range(BINS // 16):
                psum = psum + jnp.sum(hp[cc])
                gsum = gsum + jnp.sum(hg[cc])
            pv = jnp.full((16,), psum, _f32)
            gv = jnp.full((16,), gsum, _f32)
            li = _f32(0.0)
            for cc in range(BINS // 16):
                wgt = wbuf[pl.ds(cc * 16, 16)]
                diff = jnp.abs(hp[cc] * gv - hg[cc] * pv) * wgt
                li = li + jnp.sum(diff)
            pg = pv * gv
            q0 = jnp.full((16,), 1.0, _f32) / pg
            q0 = q0 * (_f32(2.0) - pg * q0)
            q0 = q0 * (_f32(2.0) - pg * q0)
            loss = loss + jnp.full((16,), li * _f32(1.0 / BINS), _f32) * q0
        locv[pl.ds(0, 16)] = loss
        pltpu.sync_copy(locv, out_hbm.at[c])


def kernel(pred_grad, gt_grad):
    pred2 = pred_grad.reshape(4 * NSUB, CH)
    gt2 = gt_grad.reshape(4 * NSUB, CH)
    mesh = plsc.VectorSubcoreMesh(core_axis_name="c", subcore_axis_name="s")
    k = pl.kernel(
        _body,
        out_type=jax.ShapeDtypeStruct((2, 16), _f32),
        mesh=mesh,
        compiler_params=pltpu.CompilerParams(needs_layout_passes=False),
        scratch_types=[
            pltpu.VMEM((CH,), _f32),
            pltpu.VMEM((CH,), _f32),
            pltpu.VMEM((CH,), _f32),
            pltpu.VMEM((CH,), _f32),
            pltpu.VMEM((8192,), _i32),
            pltpu.VMEM((4096,), _f32),
            pltpu.VMEM((32, 16), _i32),
            pltpu.VMEM((32, 16), _i32),
            pltpu.VMEM((16, 16), _f32),
            pltpu.VMEM((16, 16), _f32),
            pltpu.VMEM((128, 16), _i32),
            pltpu.VMEM((16,), _f32),
            pltpu.VMEM((2, 16), _f32),
            pltpu.VMEM((BINS,), _f32),
            pltpu.VMEM((256,), _f32),
            pltpu.VMEM_SHARED((128, 16), _i32),
            pltpu.VMEM_SHARED((16, 16), _f32),
            pltpu.VMEM_SHARED((512,), _f32),
            pltpu.SemaphoreType.DMA,
            pltpu.SemaphoreType.DMA,
            pltpu.SemaphoreType.DMA,
            pltpu.SemaphoreType.DMA,
        ],
    )
    out = k(pred2, gt2, jnp.asarray(_W64))
    return (out[0, 0] + out[1, 0]) * _f32(0.25)


# speedup vs baseline: 1.5217x; 1.0205x over previous
import jax
import jax.numpy as jnp
import numpy as np
from jax import lax
from jax.experimental import pallas as pl
from jax.experimental.pallas import tpu as pltpu
from jax.experimental.pallas import tpu_sc as plsc

BINS = 64
MARGIN = 0.4
N = 512 * 512
NSUB = 16
CH = N // NSUB
K_RANK = int(np.float32(0.95) * np.float32(N - 1))
Q_FRAC = float(np.float32(0.95) * np.float32(N - 1)) - K_RANK

_f32 = jnp.float32
_i32 = jnp.int32

_W64 = np.exp(np.float32(MARGIN) * np.arange(BINS, dtype=np.float32)
              / np.float32(BINS)).astype(np.float32)


def _body(pred_hbm, gt_hbm, wts_hbm, out_hbm,
          gt0, gt1, pd0, pd1, cnt, hist2, loc2d, rb2d, h2d, rbh2d,
          zacc, locv, minv, wbuf, rbmin,
          s_acc, s_hacc, s_min,
          sem_g0, sem_g1, sem_p0, sem_p1):
    c = lax.axis_index("c")
    s = lax.axis_index("s")
    lane = lax.iota(_i32, 16)
    ones_i = jnp.full((16,), 1, _i32)
    zeros_i = jnp.full((16,), 0, _i32)
    zeros_f = jnp.full((16,), 0.0, _f32)
    inf_v = jnp.full((16,), jnp.inf, _f32)
    lane512 = lane * 512

    row0 = (2 * c + 0) * NSUB + s
    row1 = (2 * c + 1) * NSUB + s
    cp_g0 = pltpu.async_copy(gt_hbm.at[row0], gt0, sem_g0)
    cp_g1 = pltpu.async_copy(gt_hbm.at[row1], gt1, sem_g1)
    cp_p0 = pltpu.async_copy(pred_hbm.at[row0], pd0, sem_p0)
    cp_p1 = pltpu.async_copy(pred_hbm.at[row1], pd1, sem_p1)

    @plsc.parallel_loop(0, 4096, step=16, unroll=4, carry=_i32(0))
    def _(kk, cy):
        cnt[pl.ds(kk, 16)] = zeros_i
        cnt[pl.ds(4096 + kk, 16)] = zeros_i
        hist2[pl.ds(kk, 16)] = zeros_f
        hist2[pl.ds(4096 + kk, 16)] = zeros_f
        return cy

    @plsc.parallel_loop(0, 128, step=1, unroll=4, carry=_i32(0))
    def _(rr, cy):
        zacc[rr] = zeros_i
        return cy

    for rr in range(32):
        h2d[rr] = zeros_f

    @pl.when(s == 0)
    def _():
        pltpu.sync_copy(zacc, s_acc)
        pltpu.sync_copy(h2d, s_hacc)
        pltpu.sync_copy(wts_hbm, wbuf)
    plsc.subcore_barrier()

    cp_g0.wait()
    cp_g1.wait()

    prefix = [_i32(0), _i32(0)]
    count_before = [_i32(0), _i32(0)]
    c_le = [_i32(0), _i32(0)]
    nminb = [_i32(256), _i32(256)]
    binlow = [_i32(0), _i32(0)]
    for p in range(4):
        sh = 24 - 8 * p
        pfx0, pfx1 = prefix[0], prefix[1]

        @plsc.parallel_loop(0, CH, step=16, unroll=4, carry=_i32(0))
        def _(kk, cy):
            for gbuf, off, pfx in ((gt0, 0, pfx0), (gt1, 256, pfx1)):
                v = gbuf[pl.ds(kk, 16)]
                bits = lax.bitcast_convert_type(v, _i32)
                if p == 0:
                    b = lax.shift_right_logical(bits, 24)
                    plsc.addupdate_scatter(cnt, [lane512 + (off + b)], ones_i)
                else:
                    b = lax.shift_right_logical(bits, sh) & 255
                    m = lax.shift_right_logical(bits, sh + 8) == pfx
                    plsc.addupdate_scatter(cnt, [lane512 + (off + b)], ones_i,
                                           mask=m)
            return cy

        @plsc.parallel_loop(0, 32, step=1, unroll=2, carry=_i32(0))
        def _(cc, cy):
            acc = zeros_i
            for l in range(16):
                acc = acc + cnt[pl.ds(l * 512 + cc * 16, 16)]
                cnt[pl.ds(l * 512 + cc * 16, 16)] = zeros_i
            loc2d[cc] = acc
            return cy

        slot = p * 32
        pltpu.sync_copy(loc2d.at[pl.ds(0, 16)], s_acc.at[slot + lane],
                        add=True)
        pltpu.sync_copy(loc2d.at[pl.ds(16, 16)], s_acc.at[slot + 16 + lane],
                        add=True)
        plsc.subcore_barrier()
        pltpu.sync_copy(s_acc.at[pl.ds(slot, 32)], rb2d)

        for img in range(2):
            r_loc = K_RANK - count_before[img]
            base = img * 16

            def select(cc, carry, base=base, r_loc=r_loc):
                done, bin_, running, cb, cle = carry
                h = rb2d[base + cc]
                s_inc = plsc.cumsum(h)
                tot = jnp.sum(h)
                crossed = (running + s_inc) >= (r_loc + 1)
                anyc = jnp.sum(jnp.where(crossed, 1, 0)) > 0
                nfalse = jnp.sum(jnp.where(crossed, 0, 1))
                e_inc = jnp.sum(jnp.where(lane == nfalse, s_inc, 0))
                e_exc = e_inc - jnp.sum(jnp.where(lane == nfalse, h, 0))
                hit = jnp.logical_and(done == 0, anyc)
                bin_ = jnp.where(hit, cc * 16 + nfalse, bin_)
                cle = jnp.where(hit, running + e_inc, cle)
                cb = jnp.where(hit, running + e_exc, cb)
                done = jnp.where(hit, _i32(1), done)
                return done, bin_, running + tot, cb, cle

            _, bin_, _, cb, cle = lax.fori_loop(
                0, 16, select, (_i32(0), _i32(0), _i32(0), _i32(0), _i32(0)))
            count_before[img] = cb
            c_le[img] = cle
            binlow[img] = bin_

            if p == 3:
                def nxt(cc, nm, base=base, bin_=bin_):
                    h = rb2d[base + cc]
                    idxb = cc * 16 + lane
                    cand = jnp.where(
                        jnp.logical_and(idxb > bin_, h > 0), idxb, 256)
                    return jnp.minimum(nm, jnp.min(cand))

                nminb[img] = lax.fori_loop(0, 16, nxt, _i32(256))
            prefix[img] = (prefix[img] << 8) | bin_ if p > 0 else bin_

    vk = [lax.bitcast_convert_type(jnp.full((16,), prefix[i], _i32), _f32)
          for i in range(2)]
    scan_needed = [jnp.logical_and(c_le[i] < K_RANK + 2, nminb[i] >= 256)
                   for i in range(2)]

    for img in range(2):
        gbuf = (gt0, gt1)[img]
        vki = vk[img]

        @pl.when(scan_needed[img])
        def _(gbuf=gbuf, vki=vki, img=img):
            @plsc.parallel_loop(0, CH, step=16, unroll=8, carry=inf_v)
            def mm(kk, m):
                v = gbuf[pl.ds(kk, 16)]
                return jnp.minimum(m, jnp.where(v > vki, v, inf_v))
            minv[img] = mm
            pltpu.sync_copy(minv.at[img],
                            s_min.at[pl.ds(img * 256 + s * 16, 16)])
    plsc.subcore_barrier()

    for img in range(2):
        @pl.when(scan_needed[img])
        def _(img=img):
            pltpu.sync_copy(s_min.at[pl.ds(img * 256, 256)], rbmin)
            gmv = inf_v
            for w in range(NSUB):
                gmv = jnp.minimum(gmv, rbmin[pl.ds(w * 16, 16)])
            minv[img] = jnp.full((16,), jnp.min(gmv), _f32)

    scales = []
    for img in range(2):
        vnext_pfx = lax.bitcast_convert_type(
            jnp.full((16,), prefix[img] + (nminb[img] - binlow[img]), _i32),
            _f32)
        have_dup = jnp.full((16,), c_le[img], _i32) >= (K_RANK + 2)
        in_pfx = jnp.full((16,), nminb[img], _i32) <= 255
        vnext = jnp.where(have_dup, vk[img],
                          jnp.where(in_pfx, vnext_pfx, minv[img]))
        max_val = vk[img] + _f32(Q_FRAC) * (vnext - vk[img])
        r0 = jnp.full((16,), 1.0, _f32) / max_val
        r0 = r0 * (_f32(2.0) - max_val * r0)
        r0 = r0 * (_f32(2.0) - max_val * r0)
        scales.append(_f32(BINS) * r0)

    cp_p0.wait()
    cp_p1.wait()
    ones_f = jnp.full((16,), 1.0, _f32)
    sc0, sc1 = scales

    @plsc.parallel_loop(0, CH, step=16, unroll=2, carry=_i32(0))
    def _(kk, cy):
        for buf, off, sc in ((pd0, 0, sc0), (gt0, 128, sc0),
                             (pd1, 256, sc1), (gt1, 384, sc1)):
            v = buf[pl.ds(kk, 16)]
            t = v * sc
            j = t.astype(_i32)
            fr = t - j.astype(_f32)
            m = t < _f32(64.0)
            idx = lane512 + (off + j)
            plsc.addupdate_scatter(hist2, [idx], ones_f, mask=m)
            plsc.addupdate_scatter(hist2, [idx + 64], fr, mask=m)
        return cy

    @plsc.parallel_loop(0, 32, step=1, unroll=2, carry=_i32(0))
    def _(cc, cy):
        acc = zeros_f
        for l in range(16):
            acc = acc + hist2[pl.ds(l * 512 + cc * 16, 16)]
        h2d[cc] = acc
        return cy

    pltpu.sync_copy(h2d.at[pl.ds(0, 16)], s_hacc.at[lane], add=True)
    pltpu.sync_copy(h2d.at[pl.ds(16, 16)], s_hacc.at[16 + lane], add=True)
    plsc.subcore_barrier()

    @pl.when(s == 0)
    def _():
        pltpu.sync_copy(s_hacc, rbh2d)

        def tri_hist(tgt):
            fbase = (tgt * 8 + 4) * 16
            h = []
            for cdx in range(BINS // 16):
                a = rbh2d[tgt * 8 + cdx]
                f = rbh2d[tgt * 8 + 4 + cdx]
                gpos = fbase + cdx * 16 + lane - 1
                fs = plsc.load_gather(
                    rbh2d, [lax.shift_right_logical(gpos, 4), gpos & 15])
                if cdx == 0:
                    fs = jnp.where(lane == 0, zeros_f, fs)
                h.append(a - f + fs)
            return h

        loss = zeros_f
        for img in range(2):
            hp = tri_hist(img * 2 + 0)
            hg = tri_hist(img * 2 + 1)
            psum = _f32(0.0)
            gsum = _f32(0.0)
            for cc in range(BINS // 16):
                psum = psum + jnp.sum(hp[cc])
                gsum = gsum + jnp.sum(hg[cc])
            pv = jnp.full((16,), psum, _f32)
            gv = jnp.full((16,), gsum, _f32)
            li = _f32(0.0)
            for cc in range(BINS // 16):
                wgt = wbuf[pl.ds(cc * 16, 16)]
                diff = jnp.abs(hp[cc] * gv - hg[cc] * pv) * wgt
                li = li + jnp.sum(diff)
            pg = pv * gv
            q0 = jnp.full((16,), 1.0, _f32) / pg
            q0 = q0 * (_f32(2.0) - pg * q0)
            q0 = q0 * (_f32(2.0) - pg * q0)
            loss = loss + jnp.full((16,), li * _f32(1.0 / BINS), _f32) * q0
        locv[pl.ds(0, 16)] = loss
        pltpu.sync_copy(locv, out_hbm.at[c])


def kernel(pred_grad, gt_grad):
    pred2 = pred_grad.reshape(4 * NSUB, CH)
    gt2 = gt_grad.reshape(4 * NSUB, CH)
    mesh = plsc.VectorSubcoreMesh(core_axis_name="c", subcore_axis_name="s")
    k = pl.kernel(
        _body,
        out_type=jax.ShapeDtypeStruct((2, 16), _f32),
        mesh=mesh,
        compiler_params=pltpu.CompilerParams(needs_layout_passes=False),
        scratch_types=[
            pltpu.VMEM((CH,), _f32),
            pltpu.VMEM((CH,), _f32),
            pltpu.VMEM((CH,), _f32),
            pltpu.VMEM((CH,), _f32),
            pltpu.VMEM((8192,), _i32),
            pltpu.VMEM((8192,), _f32),
            pltpu.VMEM((32, 16), _i32),
            pltpu.VMEM((32, 16), _i32),
            pltpu.VMEM((32, 16), _f32),
            pltpu.VMEM((32, 16), _f32),
            pltpu.VMEM((128, 16), _i32),
            pltpu.VMEM((16,), _f32),
            pltpu.VMEM((2, 16), _f32),
            pltpu.VMEM((BINS,), _f32),
            pltpu.VMEM((256,), _f32),
            pltpu.VMEM_SHARED((128, 16), _i32),
            pltpu.VMEM_SHARED((32, 16), _f32),
            pltpu.VMEM_SHARED((512,), _f32),
            pltpu.SemaphoreType.DMA,
            pltpu.SemaphoreType.DMA,
            pltpu.SemaphoreType.DMA,
            pltpu.SemaphoreType.DMA,
        ],
    )
    out = k(pred2, gt2, jnp.asarray(_W64))
    return (out[0, 0] + out[1, 0]) * _f32(0.25)


# speedup vs baseline: 1.6203x; 1.0648x over previous
import jax
import jax.numpy as jnp
import numpy as np
from jax import lax
from jax.experimental import pallas as pl
from jax.experimental.pallas import tpu as pltpu
from jax.experimental.pallas import tpu_sc as plsc

BINS = 64
MARGIN = 0.4
N = 512 * 512
NSUB = 16
CH = N // NSUB
K_RANK = int(np.float32(0.95) * np.float32(N - 1))
Q_FRAC = float(np.float32(0.95) * np.float32(N - 1)) - K_RANK

_f32 = jnp.float32
_i32 = jnp.int32

_W64 = np.exp(np.float32(MARGIN) * np.arange(BINS, dtype=np.float32)
              / np.float32(BINS)).astype(np.float32)


def _body(pred_hbm, gt_hbm, wts_hbm, out_hbm,
          gt0, gt1, pd0, pd1, cnt, hist2, loc2d, rb2d, h2d, rbh2d,
          zacc, locv, minv, wbuf, rbmin,
          s_acc, s_hacc, s_min,
          sem_g0, sem_g1, sem_p0, sem_p1):
    c = lax.axis_index("c")
    s = lax.axis_index("s")
    lane = lax.iota(_i32, 16)
    ones_i = jnp.full((16,), 1, _i32)
    zeros_i = jnp.full((16,), 0, _i32)
    zeros_f = jnp.full((16,), 0.0, _f32)
    inf_v = jnp.full((16,), jnp.inf, _f32)
    lane512 = lane * 512

    row0 = (2 * c + 0) * NSUB + s
    row1 = (2 * c + 1) * NSUB + s
    cp_g0 = pltpu.async_copy(gt_hbm.at[row0], gt0, sem_g0)
    cp_g1 = pltpu.async_copy(gt_hbm.at[row1], gt1, sem_g1)

    @plsc.parallel_loop(0, 4096, step=16, unroll=4, carry=_i32(0))
    def _(kk, cy):
        cnt[pl.ds(kk, 16)] = zeros_i
        cnt[pl.ds(4096 + kk, 16)] = zeros_i
        hist2[pl.ds(kk, 16)] = zeros_f
        hist2[pl.ds(4096 + kk, 16)] = zeros_f
        return cy

    @plsc.parallel_loop(0, 128, step=1, unroll=4, carry=_i32(0))
    def _(rr, cy):
        zacc[rr] = zeros_i
        return cy

    for rr in range(32):
        h2d[rr] = zeros_f

    @pl.when(s == 0)
    def _():
        pltpu.sync_copy(zacc, s_acc)
        pltpu.sync_copy(h2d, s_hacc)
        pltpu.sync_copy(wts_hbm, wbuf)
    plsc.subcore_barrier()

    cp_g0.wait()
    cp_g1.wait()

    prefix = [_i32(0), _i32(0)]
    count_before = [_i32(0), _i32(0)]
    c_le = [_i32(0), _i32(0)]
    nminb = [_i32(256), _i32(256)]
    binlow = [_i32(0), _i32(0)]
    ncomp = [_i32(0), _i32(0)]
    for p in range(4):
        sh = 24 - 8 * p
        pfx0, pfx1 = prefix[0], prefix[1]

        if p == 0:
            @plsc.parallel_loop(0, CH, step=16, unroll=4, carry=_i32(0))
            def _(kk, cy):
                for gbuf, off in ((gt0, 0), (gt1, 256)):
                    v = gbuf[pl.ds(kk, 16)]
                    bits = lax.bitcast_convert_type(v, _i32)
                    b = lax.shift_right_logical(bits, 24)
                    plsc.addupdate_scatter(cnt, [lane512 + (off + b)], ones_i)
                return cy
        elif p == 1:
            @plsc.parallel_loop(0, CH, step=16, unroll=4,
                                carry=(_i32(0), _i32(0)))
            def nloop(kk, cy):
                nb = list(cy)
                for img, gbuf, cbuf, off, pfx in (
                        (0, gt0, pd0, 0, pfx0), (1, gt1, pd1, 256, pfx1)):
                    v = gbuf[pl.ds(kk, 16)]
                    bits = lax.bitcast_convert_type(v, _i32)
                    b = lax.shift_right_logical(bits, sh) & 255
                    m = lax.shift_right_logical(bits, sh + 8) == pfx
                    plsc.addupdate_scatter(cnt, [lane512 + (off + b)], ones_i,
                                           mask=m)
                    mi = jnp.where(m, 1, 0)
                    cs = plsc.cumsum(mi)
                    plsc.store_scatter(cbuf, [nb[img] - 1 + cs], v, mask=m)
                    nb[img] = nb[img] + jnp.sum(mi)
                return tuple(nb)

            ncomp = list(nloop)
        else:
            for img, cbuf, off, pfx in ((0, pd0, 0, pfx0),
                                        (1, pd1, 256, pfx1)):
                trips = lax.shift_right_logical(ncomp[img] + 15, 4)
                nv = jnp.full((16,), ncomp[img], _i32)

                def cbody(i, cy, cbuf=cbuf, off=off, pfx=pfx, nv=nv):
                    bits = lax.bitcast_convert_type(
                        cbuf[pl.ds(i * 16, 16)], _i32)
                    b = lax.shift_right_logical(bits, sh) & 255
                    m = jnp.logical_and(
                        lax.shift_right_logical(bits, sh + 8) == pfx,
                        i * 16 + lane < nv)
                    plsc.addupdate_scatter(cnt, [lane512 + (off + b)], ones_i,
                                           mask=m)
                    return cy

                lax.fori_loop(0, trips, cbody, _i32(0))
            if p == 3:
                cp_p0 = pltpu.async_copy(pred_hbm.at[row0], pd0, sem_p0)
                cp_p1 = pltpu.async_copy(pred_hbm.at[row1], pd1, sem_p1)

        @plsc.parallel_loop(0, 32, step=1, unroll=2, carry=_i32(0))
        def _(cc, cy):
            acc = zeros_i
            for l in range(16):
                acc = acc + cnt[pl.ds(l * 512 + cc * 16, 16)]
                cnt[pl.ds(l * 512 + cc * 16, 16)] = zeros_i
            loc2d[cc] = acc
            return cy

        slot = p * 32
        pltpu.sync_copy(loc2d.at[pl.ds(0, 16)], s_acc.at[slot + lane],
                        add=True)
        pltpu.sync_copy(loc2d.at[pl.ds(16, 16)], s_acc.at[slot + 16 + lane],
                        add=True)
        plsc.subcore_barrier()
        pltpu.sync_copy(s_acc.at[pl.ds(slot, 32)], rb2d)

        for img in range(2):
            r_loc = K_RANK - count_before[img]
            base = img * 16

            def select(cc, carry, base=base, r_loc=r_loc):
                done, bin_, running, cb, cle = carry
                h = rb2d[base + cc]
                s_inc = plsc.cumsum(h)
                tot = jnp.sum(h)
                crossed = (running + s_inc) >= (r_loc + 1)
                anyc = jnp.sum(jnp.where(crossed, 1, 0)) > 0
                nfalse = jnp.sum(jnp.where(crossed, 0, 1))
                e_inc = jnp.sum(jnp.where(lane == nfalse, s_inc, 0))
                e_exc = e_inc - jnp.sum(jnp.where(lane == nfalse, h, 0))
                hit = jnp.logical_and(done == 0, anyc)
                bin_ = jnp.where(hit, cc * 16 + nfalse, bin_)
                cle = jnp.where(hit, running + e_inc, cle)
                cb = jnp.where(hit, running + e_exc, cb)
                done = jnp.where(hit, _i32(1), done)
                return done, bin_, running + tot, cb, cle

            _, bin_, _, cb, cle = lax.fori_loop(
                0, 16, select, (_i32(0), _i32(0), _i32(0), _i32(0), _i32(0)))
            count_before[img] = cb
            c_le[img] = cle
            binlow[img] = bin_

            if p == 3:
                def nxt(cc, nm, base=base, bin_=bin_):
                    h = rb2d[base + cc]
                    idxb = cc * 16 + lane
                    cand = jnp.where(
                        jnp.logical_and(idxb > bin_, h > 0), idxb, 256)
                    return jnp.minimum(nm, jnp.min(cand))

                nminb[img] = lax.fori_loop(0, 16, nxt, _i32(256))
            prefix[img] = (prefix[img] << 8) | bin_ if p > 0 else bin_

    vk = [lax.bitcast_convert_type(jnp.full((16,), prefix[i], _i32), _f32)
          for i in range(2)]
    scan_needed = [jnp.logical_and(c_le[i] < K_RANK + 2, nminb[i] >= 256)
                   for i in range(2)]

    for img in range(2):
        gbuf = (gt0, gt1)[img]
        vki = vk[img]

        @pl.when(scan_needed[img])
        def _(gbuf=gbuf, vki=vki, img=img):
            @plsc.parallel_loop(0, CH, step=16, unroll=8, carry=inf_v)
            def mm(kk, m):
                v = gbuf[pl.ds(kk, 16)]
                return jnp.minimum(m, jnp.where(v > vki, v, inf_v))
            minv[img] = mm
            pltpu.sync_copy(minv.at[img],
                            s_min.at[pl.ds(img * 256 + s * 16, 16)])
    plsc.subcore_barrier()

    for img in range(2):
        @pl.when(scan_needed[img])
        def _(img=img):
            pltpu.sync_copy(s_min.at[pl.ds(img * 256, 256)], rbmin)
            gmv = inf_v
            for w in range(NSUB):
                gmv = jnp.minimum(gmv, rbmin[pl.ds(w * 16, 16)])
            minv[img] = jnp.full((16,), jnp.min(gmv), _f32)

    scales = []
    for img in range(2):
        vnext_pfx = lax.bitcast_convert_type(
            jnp.full((16,), prefix[img] + (nminb[img] - binlow[img]), _i32),
            _f32)
        have_dup = jnp.full((16,), c_le[img], _i32) >= (K_RANK + 2)
        in_pfx = jnp.full((16,), nminb[img], _i32) <= 255
        vnext = jnp.where(have_dup, vk[img],
                          jnp.where(in_pfx, vnext_pfx, minv[img]))
        max_val = vk[img] + _f32(Q_FRAC) * (vnext - vk[img])
        r0 = jnp.full((16,), 1.0, _f32) / max_val
        r0 = r0 * (_f32(2.0) - max_val * r0)
        r0 = r0 * (_f32(2.0) - max_val * r0)
        scales.append(_f32(BINS) * r0)

    cp_p0.wait()
    cp_p1.wait()
    ones_f = jnp.full((16,), 1.0, _f32)
    sc0, sc1 = scales

    @plsc.parallel_loop(0, CH, step=16, unroll=2, carry=_i32(0))
    def _(kk, cy):
        for buf, off, sc in ((pd0, 0, sc0), (gt0, 128, sc0),
                             (pd1, 256, sc1), (gt1, 384, sc1)):
            v = buf[pl.ds(kk, 16)]
            t = v * sc
            j = t.astype(_i32)
            fr = t - j.astype(_f32)
            m = t < _f32(64.0)
            idx = lane512 + (off + j)
            plsc.addupdate_scatter(hist2, [idx], ones_f, mask=m)
            plsc.addupdate_scatter(hist2, [idx + 64], fr, mask=m)
        return cy

    @plsc.parallel_loop(0, 32, step=1, unroll=2, carry=_i32(0))
    def _(cc, cy):
        acc = zeros_f
        for l in range(16):
            acc = acc + hist2[pl.ds(l * 512 + cc * 16, 16)]
        h2d[cc] = acc
        return cy

    pltpu.sync_copy(h2d.at[pl.ds(0, 16)], s_hacc.at[lane], add=True)
    pltpu.sync_copy(h2d.at[pl.ds(16, 16)], s_hacc.at[16 + lane], add=True)
    plsc.subcore_barrier()

    @pl.when(s == 0)
    def _():
        pltpu.sync_copy(s_hacc, rbh2d)

        def tri_hist(tgt):
            fbase = (tgt * 8 + 4) * 16
            h = []
            for cdx in range(BINS // 16):
                a = rbh2d[tgt * 8 + cdx]
                f = rbh2d[tgt * 8 + 4 + cdx]
                gpos = fbase + cdx * 16 + lane - 1
                fs = plsc.load_gather(
                    rbh2d, [lax.shift_right_logical(gpos, 4), gpos & 15])
                if cdx == 0:
                    fs = jnp.where(lane == 0, zeros_f, fs)
                h.append(a - f + fs)
            return h

        loss = zeros_f
        for img in range(2):
            hp = tri_hist(img * 2 + 0)
            hg = tri_hist(img * 2 + 1)
            psum = _f32(0.0)
            gsum = _f32(0.0)
            for cc in range(BINS // 16):
                psum = psum + jnp.sum(hp[cc])
                gsum = gsum + jnp.sum(hg[cc])
            pv = jnp.full((16,), psum, _f32)
            gv = jnp.full((16,), gsum, _f32)
            li = _f32(0.0)
            for cc in range(BINS // 16):
                wgt = wbuf[pl.ds(cc * 16, 16)]
                diff = jnp.abs(hp[cc] * gv - hg[cc] * pv) * wgt
                li = li + jnp.sum(diff)
            pg = pv * gv
            q0 = jnp.full((16,), 1.0, _f32) / pg
            q0 = q0 * (_f32(2.0) - pg * q0)
            q0 = q0 * (_f32(2.0) - pg * q0)
            loss = loss + jnp.full((16,), li * _f32(1.0 / BINS), _f32) * q0
        locv[pl.ds(0, 16)] = loss
        pltpu.sync_copy(locv, out_hbm.at[c])


def kernel(pred_grad, gt_grad):
    pred2 = pred_grad.reshape(4 * NSUB, CH)
    gt2 = gt_grad.reshape(4 * NSUB, CH)
    mesh = plsc.VectorSubcoreMesh(core_axis_name="c", subcore_axis_name="s")
    k = pl.kernel(
        _body,
        out_type=jax.ShapeDtypeStruct((2, 16), _f32),
        mesh=mesh,
        compiler_params=pltpu.CompilerParams(needs_layout_passes=False),
        scratch_types=[
            pltpu.VMEM((CH,), _f32),
            pltpu.VMEM((CH,), _f32),
            pltpu.VMEM((CH,), _f32),
            pltpu.VMEM((CH,), _f32),
            pltpu.VMEM((8192,), _i32),
            pltpu.VMEM((8192,), _f32),
            pltpu.VMEM((32, 16), _i32),
            pltpu.VMEM((32, 16), _i32),
            pltpu.VMEM((32, 16), _f32),
            pltpu.VMEM((32, 16), _f32),
            pltpu.VMEM((128, 16), _i32),
            pltpu.VMEM((16,), _f32),
            pltpu.VMEM((2, 16), _f32),
            pltpu.VMEM((BINS,), _f32),
            pltpu.VMEM((256,), _f32),
            pltpu.VMEM_SHARED((128, 16), _i32),
            pltpu.VMEM_SHARED((32, 16), _f32),
            pltpu.VMEM_SHARED((512,), _f32),
            pltpu.SemaphoreType.DMA,
            pltpu.SemaphoreType.DMA,
            pltpu.SemaphoreType.DMA,
            pltpu.SemaphoreType.DMA,
        ],
    )
    out = k(pred2, gt2, jnp.asarray(_W64))
    return (out[0, 0] + out[1, 0]) * _f32(0.25)


# speedup vs baseline: 1.6229x; 1.0016x over previous
import jax
import jax.numpy as jnp
import numpy as np
from jax import lax
from jax.experimental import pallas as pl
from jax.experimental.pallas import tpu as pltpu
from jax.experimental.pallas import tpu_sc as plsc

BINS = 64
MARGIN = 0.4
N = 512 * 512
NSUB = 16
CH = N // NSUB
K_RANK = int(np.float32(0.95) * np.float32(N - 1))
Q_FRAC = float(np.float32(0.95) * np.float32(N - 1)) - K_RANK

_f32 = jnp.float32
_i32 = jnp.int32

_W64 = np.exp(np.float32(MARGIN) * np.arange(BINS, dtype=np.float32)
              / np.float32(BINS)).astype(np.float32)


def _body(pred_hbm, gt_hbm, wts_hbm, out_hbm,
          gt0, gt1, pd0, pd1, cnt, hist2, loc2d, rb2d, h2d, rbh2d,
          zacc, locv, minv, wbuf, rbmin,
          s_acc, s_hacc, s_min,
          sem_g0, sem_g1, sem_p0, sem_p1):
    c = lax.axis_index("c")
    s = lax.axis_index("s")
    lane = lax.iota(_i32, 16)
    ones_i = jnp.full((16,), 1, _i32)
    zeros_i = jnp.full((16,), 0, _i32)
    zeros_f = jnp.full((16,), 0.0, _f32)
    inf_v = jnp.full((16,), jnp.inf, _f32)
    lane512 = lane * 512

    row0 = (2 * c + 0) * NSUB + s
    row1 = (2 * c + 1) * NSUB + s
    cp_g0 = pltpu.async_copy(gt_hbm.at[row0], gt0, sem_g0)
    cp_g1 = pltpu.async_copy(gt_hbm.at[row1], gt1, sem_g1)

    @plsc.parallel_loop(0, 4096, step=16, unroll=4, carry=_i32(0))
    def _(kk, cy):
        cnt[pl.ds(kk, 16)] = zeros_i
        cnt[pl.ds(4096 + kk, 16)] = zeros_i
        hist2[pl.ds(kk, 16)] = zeros_f
        hist2[pl.ds(4096 + kk, 16)] = zeros_f
        return cy

    @plsc.parallel_loop(0, 128, step=1, unroll=4, carry=_i32(0))
    def _(rr, cy):
        zacc[rr] = zeros_i
        return cy

    for rr in range(32):
        h2d[rr] = zeros_f

    @pl.when(s == 0)
    def _():
        pltpu.sync_copy(zacc, s_acc)
        pltpu.sync_copy(h2d, s_hacc)
        pltpu.sync_copy(wts_hbm, wbuf)
    plsc.subcore_barrier()

    cp_g0.wait()
    cp_g1.wait()

    prefix = [_i32(0), _i32(0)]
    count_before = [_i32(0), _i32(0)]
    c_le = [_i32(0), _i32(0)]
    nminb = [_i32(256), _i32(256)]
    binlow = [_i32(0), _i32(0)]
    ncomp = [_i32(0), _i32(0)]
    for p in range(4):
        sh = 24 - 8 * p
        pfx0, pfx1 = prefix[0], prefix[1]

        if p == 0:
            @plsc.parallel_loop(0, CH, step=16, unroll=4, carry=_i32(0))
            def _(kk, cy):
                for gbuf, off in ((gt0, 0), (gt1, 256)):
                    v = gbuf[pl.ds(kk, 16)]
                    bits = lax.bitcast_convert_type(v, _i32)
                    b = lax.shift_right_logical(bits, 24)
                    plsc.addupdate_scatter(cnt, [lane512 + (off + b)], ones_i)
                return cy
        elif p == 1:
            @plsc.parallel_loop(0, CH, step=16, unroll=4,
                                carry=(_i32(0), _i32(0)))
            def nloop(kk, cy):
                nb = list(cy)
                for img, gbuf, cbuf, off, pfx in (
                        (0, gt0, pd0, 0, pfx0), (1, gt1, pd1, 256, pfx1)):
                    v = gbuf[pl.ds(kk, 16)]
                    bits = lax.bitcast_convert_type(v, _i32)
                    m = lax.shift_right_logical(bits, sh + 8) == pfx
                    mi = jnp.where(m, 1, 0)
                    cs = plsc.cumsum(mi)
                    plsc.store_scatter(cbuf, [nb[img] - 1 + cs], v, mask=m)
                    nb[img] = nb[img] + jnp.sum(mi)
                return tuple(nb)

            ncomp = list(nloop)

            for img, cbuf, off in ((0, pd0, 0), (1, pd1, 256)):
                trips = lax.shift_right_logical(ncomp[img] + 15, 4)
                nv = jnp.full((16,), ncomp[img], _i32)

                def hbody(i, cy, cbuf=cbuf, off=off, nv=nv):
                    bits = lax.bitcast_convert_type(
                        cbuf[pl.ds(i * 16, 16)], _i32)
                    b = lax.shift_right_logical(bits, sh) & 255
                    plsc.addupdate_scatter(cnt, [lane512 + (off + b)], ones_i,
                                           mask=i * 16 + lane < nv)
                    return cy

                lax.fori_loop(0, trips, hbody, _i32(0))
        else:
            for img, cbuf, off, pfx in ((0, pd0, 0, pfx0),
                                        (1, pd1, 256, pfx1)):
                trips = lax.shift_right_logical(ncomp[img] + 15, 4)
                nv = jnp.full((16,), ncomp[img], _i32)

                def cbody(i, cy, cbuf=cbuf, off=off, pfx=pfx, nv=nv):
                    bits = lax.bitcast_convert_type(
                        cbuf[pl.ds(i * 16, 16)], _i32)
                    b = lax.shift_right_logical(bits, sh) & 255
                    m = jnp.logical_and(
                        lax.shift_right_logical(bits, sh + 8) == pfx,
                        i * 16 + lane < nv)
                    plsc.addupdate_scatter(cnt, [lane512 + (off + b)], ones_i,
                                           mask=m)
                    return cy

                lax.fori_loop(0, trips, cbody, _i32(0))
            if p == 3:
                cp_p0 = pltpu.async_copy(pred_hbm.at[row0], pd0, sem_p0)
                cp_p1 = pltpu.async_copy(pred_hbm.at[row1], pd1, sem_p1)

        @plsc.parallel_loop(0, 32, step=1, unroll=2, carry=_i32(0))
        def _(cc, cy):
            acc = zeros_i
            for l in range(16):
                acc = acc + cnt[pl.ds(l * 512 + cc * 16, 16)]
                cnt[pl.ds(l * 512 + cc * 16, 16)] = zeros_i
            loc2d[cc] = acc
            return cy

        slot = p * 32
        pltpu.sync_copy(loc2d.at[pl.ds(0, 16)], s_acc.at[slot + lane],
                        add=True)
        pltpu.sync_copy(loc2d.at[pl.ds(16, 16)], s_acc.at[slot + 16 + lane],
                        add=True)
        plsc.subcore_barrier()
        pltpu.sync_copy(s_acc.at[pl.ds(slot, 32)], rb2d)

        for img in range(2):
            r_loc = K_RANK - count_before[img]
            base = img * 16

            def select(cc, carry, base=base, r_loc=r_loc):
                done, bin_, running, cb, cle = carry
                h = rb2d[base + cc]
                s_inc = plsc.cumsum(h)
                tot = jnp.sum(h)
                crossed = (running + s_inc) >= (r_loc + 1)
                anyc = jnp.sum(jnp.where(crossed, 1, 0)) > 0
                nfalse = jnp.sum(jnp.where(crossed, 0, 1))
                e_inc = jnp.sum(jnp.where(lane == nfalse, s_inc, 0))
                e_exc = e_inc - jnp.sum(jnp.where(lane == nfalse, h, 0))
                hit = jnp.logical_and(done == 0, anyc)
                bin_ = jnp.where(hit, cc * 16 + nfalse, bin_)
                cle = jnp.where(hit, running + e_inc, cle)
                cb = jnp.where(hit, running + e_exc, cb)
                done = jnp.where(hit, _i32(1), done)
                return done, bin_, running + tot, cb, cle

            _, bin_, _, cb, cle = lax.fori_loop(
                0, 16, select, (_i32(0), _i32(0), _i32(0), _i32(0), _i32(0)))
            count_before[img] = cb
            c_le[img] = cle
            binlow[img] = bin_

            if p == 3:
                def nxt(cc, nm, base=base, bin_=bin_):
                    h = rb2d[base + cc]
                    idxb = cc * 16 + lane
                    cand = jnp.where(
                        jnp.logical_and(idxb > bin_, h > 0), idxb, 256)
                    return jnp.minimum(nm, jnp.min(cand))

                nminb[img] = lax.fori_loop(0, 16, nxt, _i32(256))
            prefix[img] = (prefix[img] << 8) | bin_ if p > 0 else bin_

    vk = [lax.bitcast_convert_type(jnp.full((16,), prefix[i], _i32), _f32)
          for i in range(2)]
    scan_needed = [jnp.logical_and(c_le[i] < K_RANK + 2, nminb[i] >= 256)
                   for i in range(2)]

    for img in range(2):
        gbuf = (gt0, gt1)[img]
        vki = vk[img]

        @pl.when(scan_needed[img])
        def _(gbuf=gbuf, vki=vki, img=img):
            @plsc.parallel_loop(0, CH, step=16, unroll=8, carry=inf_v)
            def mm(kk, m):
                v = gbuf[pl.ds(kk, 16)]
                return jnp.minimum(m, jnp.where(v > vki, v, inf_v))
            minv[img] = mm
            pltpu.sync_copy(minv.at[img],
                            s_min.at[pl.ds(img * 256 + s * 16, 16)])
    plsc.subcore_barrier()

    for img in range(2):
        @pl.when(scan_needed[img])
        def _(img=img):
            pltpu.sync_copy(s_min.at[pl.ds(img * 256, 256)], rbmin)
            gmv = inf_v
            for w in range(NSUB):
                gmv = jnp.minimum(gmv, rbmin[pl.ds(w * 16, 16)])
            minv[img] = jnp.full((16,), jnp.min(gmv), _f32)

    scales = []
    for img in range(2):
        vnext_pfx = lax.bitcast_convert_type(
            jnp.full((16,), prefix[img] + (nminb[img] - binlow[img]), _i32),
            _f32)
        have_dup = jnp.full((16,), c_le[img], _i32) >= (K_RANK + 2)
        in_pfx = jnp.full((16,), nminb[img], _i32) <= 255
        vnext = jnp.where(have_dup, vk[img],
                          jnp.where(in_pfx, vnext_pfx, minv[img]))
        max_val = vk[img] + _f32(Q_FRAC) * (vnext - vk[img])
        r0 = jnp.full((16,), 1.0, _f32) / max_val
        r0 = r0 * (_f32(2.0) - max_val * r0)
        r0 = r0 * (_f32(2.0) - max_val * r0)
        scales.append(_f32(BINS) * r0)

    cp_p0.wait()
    cp_p1.wait()
    ones_f = jnp.full((16,), 1.0, _f32)
    sc0, sc1 = scales

    @plsc.parallel_loop(0, CH, step=16, unroll=2, carry=_i32(0))
    def _(kk, cy):
        for buf, off, sc in ((pd0, 0, sc0), (gt0, 128, sc0),
                             (pd1, 256, sc1), (gt1, 384, sc1)):
            v = buf[pl.ds(kk, 16)]
            t = v * sc
            j = t.astype(_i32)
            fr = t - j.astype(_f32)
            m = t < _f32(64.0)
            idx = lane512 + (off + j)
            plsc.addupdate_scatter(hist2, [idx], ones_f, mask=m)
            plsc.addupdate_scatter(hist2, [idx + 64], fr, mask=m)
        return cy

    @plsc.parallel_loop(0, 32, step=1, unroll=2, carry=_i32(0))
    def _(cc, cy):
        acc = zeros_f
        for l in range(16):
            acc = acc + hist2[pl.ds(l * 512 + cc * 16, 16)]
        h2d[cc] = acc
        return cy

    pltpu.sync_copy(h2d.at[pl.ds(0, 16)], s_hacc.at[lane], add=True)
    pltpu.sync_copy(h2d.at[pl.ds(16, 16)], s_hacc.at[16 + lane], add=True)
    plsc.subcore_barrier()

    @pl.when(s == 0)
    def _():
        pltpu.sync_copy(s_hacc, rbh2d)

        def tri_hist(tgt):
            fbase = (tgt * 8 + 4) * 16
            h = []
            for cdx in range(BINS // 16):
                a = rbh2d[tgt * 8 + cdx]
                f = rbh2d[tgt * 8 + 4 + cdx]
                gpos = fbase + cdx * 16 + lane - 1
                fs = plsc.load_gather(
                    rbh2d, [lax.shift_right_logical(gpos, 4), gpos & 15])
                if cdx == 0:
                    fs = jnp.where(lane == 0, zeros_f, fs)
                h.append(a - f + fs)
            return h

        loss = zeros_f
        for img in range(2):
            hp = tri_hist(img * 2 + 0)
            hg = tri_hist(img * 2 + 1)
            psum = _f32(0.0)
            gsum = _f32(0.0)
            for cc in range(BINS // 16):
                psum = psum + jnp.sum(hp[cc])
                gsum = gsum + jnp.sum(hg[cc])
            pv = jnp.full((16,), psum, _f32)
            gv = jnp.full((16,), gsum, _f32)
            li = _f32(0.0)
            for cc in range(BINS // 16):
                wgt = wbuf[pl.ds(cc * 16, 16)]
                diff = jnp.abs(hp[cc] * gv - hg[cc] * pv) * wgt
                li = li + jnp.sum(diff)
            pg = pv * gv
            q0 = jnp.full((16,), 1.0, _f32) / pg
            q0 = q0 * (_f32(2.0) - pg * q0)
            q0 = q0 * (_f32(2.0) - pg * q0)
            loss = loss + jnp.full((16,), li * _f32(1.0 / BINS), _f32) * q0
        locv[pl.ds(0, 16)] = loss
        pltpu.sync_copy(locv, out_hbm.at[c])


def kernel(pred_grad, gt_grad):
    pred2 = pred_grad.reshape(4 * NSUB, CH)
    gt2 = gt_grad.reshape(4 * NSUB, CH)
    mesh = plsc.VectorSubcoreMesh(core_axis_name="c", subcore_axis_name="s")
    k = pl.kernel(
        _body,
        out_type=jax.ShapeDtypeStruct((2, 16), _f32),
        mesh=mesh,
        compiler_params=pltpu.CompilerParams(needs_layout_passes=False),
        scratch_types=[
            pltpu.VMEM((CH,), _f32),
            pltpu.VMEM((CH,), _f32),
            pltpu.VMEM((CH,), _f32),
            pltpu.VMEM((CH,), _f32),
            pltpu.VMEM((8192,), _i32),
            pltpu.VMEM((8192,), _f32),
            pltpu.VMEM((32, 16), _i32),
            pltpu.VMEM((32, 16), _i32),
            pltpu.VMEM((32, 16), _f32),
            pltpu.VMEM((32, 16), _f32),
            pltpu.VMEM((128, 16), _i32),
            pltpu.VMEM((16,), _f32),
            pltpu.VMEM((2, 16), _f32),
            pltpu.VMEM((BINS,), _f32),
            pltpu.VMEM((256,), _f32),
            pltpu.VMEM_SHARED((128, 16), _i32),
            pltpu.VMEM_SHARED((32, 16), _f32),
            pltpu.VMEM_SHARED((512,), _f32),
            pltpu.SemaphoreType.DMA,
            pltpu.SemaphoreType.DMA,
            pltpu.SemaphoreType.DMA,
            pltpu.SemaphoreType.DMA,
        ],
    )
    out = k(pred2, gt2, jnp.asarray(_W64))
    return (out[0, 0] + out[1, 0]) * _f32(0.25)
